# baseline (speedup 1.0000x reference)
import jax
import jax.numpy as jnp
from jax.experimental import pallas as pl
from jax.experimental.pallas import tpu as pltpu


def _passthrough(img_ref, lab_ref, img_out, lab_out, sem_img, sem_lab):
    cp_img = pltpu.make_async_copy(img_ref, img_out, sem_img)
    cp_lab = pltpu.make_async_copy(lab_ref, lab_out, sem_lab)
    cp_img.start()
    cp_lab.start()
    cp_img.wait()
    cp_lab.wait()


def kernel(images, labels):
    out_img, out_lab = pl.pallas_call(
        _passthrough,
        in_specs=[
            pl.BlockSpec(memory_space=pl.ANY),
            pl.BlockSpec(memory_space=pl.ANY),
        ],
        out_specs=[
            pl.BlockSpec(memory_space=pl.ANY),
            pl.BlockSpec(memory_space=pl.ANY),
        ],
        out_shape=[
            jax.ShapeDtypeStruct(images.shape, images.dtype),
            jax.ShapeDtypeStruct(labels.shape, labels.dtype),
        ],
        scratch_shapes=[pltpu.SemaphoreType.DMA, pltpu.SemaphoreType.DMA],
    )(images, labels)
    return (out_img, out_lab)


# speedup vs baseline: 36.2260x; 36.2260x over previous
import jax
import jax.numpy as jnp
from jax.experimental import pallas as pl
from jax.experimental.pallas import tpu as pltpu


def _passthrough(img_ref, lab_ref, img_out, lab_out, sem_img, sem_lab):
    cp_img = pltpu.make_async_copy(img_ref, img_out, sem_img)
    cp_lab = pltpu.make_async_copy(lab_ref, lab_out, sem_lab)
    cp_img.start()
    cp_lab.start()
    cp_img.wait()
    cp_lab.wait()


def kernel(images, labels):
    B, H, W, C = images.shape
    img2 = images.reshape(B * H, W * C)
    out_img, out_lab = pl.pallas_call(
        _passthrough,
        in_specs=[
            pl.BlockSpec(memory_space=pl.ANY),
            pl.BlockSpec(memory_space=pl.ANY),
        ],
        out_specs=[
            pl.BlockSpec(memory_space=pl.ANY),
            pl.BlockSpec(memory_space=pl.ANY),
        ],
        out_shape=[
            jax.ShapeDtypeStruct(img2.shape, img2.dtype),
            jax.ShapeDtypeStruct(labels.shape, labels.dtype),
        ],
        scratch_shapes=[pltpu.SemaphoreType.DMA, pltpu.SemaphoreType.DMA],
    )(img2, labels)
    return (out_img.reshape(B, H, W, C), out_lab)


# speedup vs baseline: 36.4356x; 1.0058x over previous
import jax
import jax.numpy as jnp
from jax.experimental import pallas as pl
from jax.experimental.pallas import tpu as pltpu


_NCHUNK = 16


def _passthrough(img_ref, lab_ref, img_out, lab_out, sems, sem_lab):
    rows = img_ref.shape[0]
    chunk = rows // _NCHUNK
    copies = []
    for i in range(_NCHUNK):
        sl = pl.ds(i * chunk, chunk)
        copies.append(
            pltpu.make_async_copy(img_ref.at[sl], img_out.at[sl], sems.at[i])
        )
    cp_lab = pltpu.make_async_copy(lab_ref, lab_out, sem_lab)
    for cp in copies:
        cp.start()
    cp_lab.start()
    for cp in copies:
        cp.wait()
    cp_lab.wait()


def kernel(images, labels):
    B, H, W, C = images.shape
    img2 = images.reshape(B * H, W * C)
    out_img, out_lab = pl.pallas_call(
        _passthrough,
        in_specs=[
            pl.BlockSpec(memory_space=pl.ANY),
            pl.BlockSpec(memory_space=pl.ANY),
        ],
        out_specs=[
            pl.BlockSpec(memory_space=pl.ANY),
            pl.BlockSpec(memory_space=pl.ANY),
        ],
        out_shape=[
            jax.ShapeDtypeStruct(img2.shape, img2.dtype),
            jax.ShapeDtypeStruct(labels.shape, labels.dtype),
        ],
        scratch_shapes=[
            pltpu.SemaphoreType.DMA((_NCHUNK,)),
            pltpu.SemaphoreType.DMA,
        ],
    )(img2, labels)
    return (out_img.reshape(B, H, W, C), out_lab)
